# baseline (speedup 1.0000x reference)
import functools

import jax
import jax.numpy as jnp
from jax import lax
from jax.experimental import pallas as pl
from jax.experimental.pallas import tpu as pltpu
from jax.experimental.pallas import tpu_sc as plsc


def _sc_gather(t, emb):
    T = t.shape[0]
    V, D = emb.shape
    info = plsc.get_sparse_core_info()
    nc, ns = info.num_cores, info.num_subcores
    nw = nc * ns
    rows_per_w = T // nw

    mesh = plsc.VectorSubcoreMesh(core_axis_name="c", subcore_axis_name="s")

    @functools.partial(
        pl.kernel,
        mesh=mesh,
        out_type=jax.ShapeDtypeStruct((T, D), jnp.float32),
        scratch_types=[
            pltpu.VMEM((rows_per_w,), jnp.int32),
            pltpu.VMEM((rows_per_w, D), jnp.float32),
            pltpu.SemaphoreType.DMA,
        ],
    )
    def gather_kernel(t_hbm, emb_hbm, out_hbm, idx_v, rows_v, sem):
        wid = lax.axis_index("s") * nc + lax.axis_index("c")
        base = wid * rows_per_w
        pltpu.sync_copy(t_hbm.at[pl.ds(base, rows_per_w)], idx_v)
        pltpu.async_copy(emb_hbm.at[idx_v], rows_v, sem).wait()
        pltpu.sync_copy(rows_v, out_hbm.at[pl.ds(base, rows_per_w)])

    return gather_kernel(t, emb)


_TR = 256
_CH = 32
_NCHUNK = _TR // _CH


def _tc_body(x_ref, e_ref, w_ref, b_ref, out_ref, te_ref):
    i = pl.program_id(0)
    j = pl.program_id(1)
    nt = pl.num_programs(0)

    def _mm(rows):
        return lax.dot_general(
            rows, w_ref[...],
            dimension_numbers=(((1,), (1,)), ((), ())),
            preferred_element_type=jnp.float32,
        ) + b_ref[...]

    @pl.when((i == 0) & (j == 0))
    def _():
        te_ref[0] = _mm(e_ref[:_TR])

    @pl.when((i + 1 < nt) & (j < _NCHUNK))
    def _():
        rs = (i + 1) * _TR + j * _CH
        te_ref[(i + 1) % 2, pl.ds(j * _CH, _CH), :] = _mm(
            e_ref[pl.ds(rs, _CH), :])

    out_ref[...] = x_ref[...] + te_ref[i % 2][None]


def _proj_add(x3, e, W, b, interpret=False):
    BH, T, D = x3.shape
    nt = T // _TR
    b2 = b.reshape(1, D)
    return pl.pallas_call(
        _tc_body,
        grid=(nt, BH),
        in_specs=[
            pl.BlockSpec((1, _TR, D), lambda i, j: (j, i, 0)),
            pl.BlockSpec((T, D), lambda i, j: (0, 0)),
            pl.BlockSpec((D, D), lambda i, j: (0, 0)),
            pl.BlockSpec((1, D), lambda i, j: (0, 0)),
        ],
        out_specs=pl.BlockSpec((1, _TR, D), lambda i, j: (j, i, 0)),
        out_shape=jax.ShapeDtypeStruct((BH, T, D), jnp.float32),
        scratch_shapes=[pltpu.VMEM((2, _TR, D), jnp.float32)],
        interpret=interpret,
    )(x3, e, W, b2)


def kernel(x, t, emb, W, b):
    B, H, T, D = x.shape
    e = _sc_gather(t, emb)
    x3 = x.reshape(B * H, T, D)
    out3 = _proj_add(x3, e, W, b)
    return out3.reshape(B, H, T, D)


# speedup vs baseline: 1.0056x; 1.0056x over previous
import functools

import jax
import jax.numpy as jnp
from jax import lax
from jax.experimental import pallas as pl
from jax.experimental.pallas import tpu as pltpu
from jax.experimental.pallas import tpu_sc as plsc


def _sc_gather(t, emb):
    T = t.shape[0]
    V, D = emb.shape
    info = plsc.get_sparse_core_info()
    nc, ns = info.num_cores, info.num_subcores
    nw = nc * ns
    rows_per_w = T // nw

    mesh = plsc.VectorSubcoreMesh(core_axis_name="c", subcore_axis_name="s")

    @functools.partial(
        pl.kernel,
        mesh=mesh,
        out_type=jax.ShapeDtypeStruct((T, D), jnp.float32),
        scratch_types=[
            pltpu.VMEM((rows_per_w,), jnp.int32),
            pltpu.VMEM((rows_per_w, D), jnp.float32),
            pltpu.SemaphoreType.DMA,
        ],
    )
    def gather_kernel(t_hbm, emb_hbm, out_hbm, idx_v, rows_v, sem):
        wid = lax.axis_index("s") * nc + lax.axis_index("c")
        base = wid * rows_per_w
        pltpu.sync_copy(t_hbm.at[pl.ds(base, rows_per_w)], idx_v)
        pltpu.async_copy(emb_hbm.at[idx_v], rows_v, sem).wait()
        pltpu.sync_copy(rows_v, out_hbm.at[pl.ds(base, rows_per_w)])

    return gather_kernel(t, emb)


_TR = 256
_CN = 128
_NCHUNK = 8


def _tc_body(x_ref, e_ref, w_ref, b_ref, out_ref, te_ref):
    i = pl.program_id(0)
    j = pl.program_id(1)
    nt = pl.num_programs(0)
    D = w_ref.shape[0]

    def _mm_cols(row0, col0, ncols):
        return lax.dot_general(
            e_ref[pl.ds(row0, _TR), :], w_ref[pl.ds(col0, ncols), :],
            dimension_numbers=(((1,), (1,)), ((), ())),
            preferred_element_type=jnp.float32,
        ) + b_ref[:, pl.ds(col0, ncols)]

    @pl.when((i == 0) & (j == 0))
    def _():
        te_ref[0] = _mm_cols(0, 0, D)

    @pl.when((i + 1 < nt) & (j < _NCHUNK))
    def _():
        te_ref[(i + 1) % 2, :, pl.ds(j * _CN, _CN)] = _mm_cols(
            (i + 1) * _TR, j * _CN, _CN)

    out_ref[...] = x_ref[...] + te_ref[i % 2][None]


def _proj_add(x3, e, W, b, interpret=False):
    BH, T, D = x3.shape
    nt = T // _TR
    b2 = b.reshape(1, D)
    return pl.pallas_call(
        _tc_body,
        grid=(nt, BH),
        in_specs=[
            pl.BlockSpec((1, _TR, D), lambda i, j: (j, i, 0)),
            pl.BlockSpec((T, D), lambda i, j: (0, 0)),
            pl.BlockSpec((D, D), lambda i, j: (0, 0)),
            pl.BlockSpec((1, D), lambda i, j: (0, 0)),
        ],
        out_specs=pl.BlockSpec((1, _TR, D), lambda i, j: (j, i, 0)),
        out_shape=jax.ShapeDtypeStruct((BH, T, D), jnp.float32),
        scratch_shapes=[pltpu.VMEM((2, _TR, D), jnp.float32)],
        interpret=interpret,
    )(x3, e, W, b2)


def kernel(x, t, emb, W, b):
    B, H, T, D = x.shape
    e = _sc_gather(t, emb)
    x3 = x.reshape(B * H, T, D)
    out3 = _proj_add(x3, e.astype(jnp.bfloat16), W.astype(jnp.bfloat16), b)
    return out3.reshape(B, H, T, D)


# speedup vs baseline: 1.3629x; 1.3553x over previous
import functools

import jax
import jax.numpy as jnp
from jax import lax
from jax.experimental import pallas as pl
from jax.experimental.pallas import tpu as pltpu
from jax.experimental.pallas import tpu_sc as plsc


def _sc_gather(t, emb):
    T = t.shape[0]
    V, D = emb.shape
    info = plsc.get_sparse_core_info()
    nc, ns = info.num_cores, info.num_subcores
    nw = nc * ns
    rows_per_w = T // nw

    mesh = plsc.VectorSubcoreMesh(core_axis_name="c", subcore_axis_name="s")

    @functools.partial(
        pl.kernel,
        mesh=mesh,
        out_type=jax.ShapeDtypeStruct((T, D), jnp.float32),
        scratch_types=[
            pltpu.VMEM((rows_per_w,), jnp.int32),
            pltpu.VMEM((rows_per_w, D), jnp.float32),
            pltpu.SemaphoreType.DMA,
        ],
    )
    def gather_kernel(t_hbm, emb_hbm, out_hbm, idx_v, rows_v, sem):
        wid = lax.axis_index("s") * nc + lax.axis_index("c")
        base = wid * rows_per_w
        pltpu.sync_copy(t_hbm.at[pl.ds(base, rows_per_w)], idx_v)
        pltpu.async_copy(emb_hbm.at[idx_v], rows_v, sem).wait()
        pltpu.sync_copy(rows_v, out_hbm.at[pl.ds(base, rows_per_w)])

    return gather_kernel(t, emb)


_TR = 512


def _tc_body(x_ref, e_ref, w_ref, b_ref, out_ref, te_ref):
    i = pl.program_id(0)
    j = pl.program_id(1)

    @pl.when((i == 0) & (j == 0))
    def _():
        te_ref[...] = lax.dot_general(
            e_ref[...], w_ref[...],
            dimension_numbers=(((1,), (1,)), ((), ())),
            preferred_element_type=jnp.float32,
        ) + b_ref[...]

    out_ref[...] = x_ref[...] + te_ref[pl.ds(j * _TR, _TR), :][None]


def _proj_add(x3, e, W, b, interpret=False):
    BH, T, D = x3.shape
    nt = T // _TR
    b2 = b.reshape(1, D)
    return pl.pallas_call(
        _tc_body,
        grid=(BH, nt),
        in_specs=[
            pl.BlockSpec((1, _TR, D), lambda i, j: (i, j, 0)),
            pl.BlockSpec((T, D), lambda i, j: (0, 0)),
            pl.BlockSpec((D, D), lambda i, j: (0, 0)),
            pl.BlockSpec((1, D), lambda i, j: (0, 0)),
        ],
        out_specs=pl.BlockSpec((1, _TR, D), lambda i, j: (i, j, 0)),
        out_shape=jax.ShapeDtypeStruct((BH, T, D), jnp.float32),
        scratch_shapes=[pltpu.VMEM((T, D), jnp.float32)],
        interpret=interpret,
    )(x3, e, W, b2)


def kernel(x, t, emb, W, b):
    B, H, T, D = x.shape
    e = _sc_gather(t, emb)
    x3 = x.reshape(B * H, T, D)
    out3 = _proj_add(x3, e.astype(jnp.bfloat16), W.astype(jnp.bfloat16), b)
    return out3.reshape(B, H, T, D)


# speedup vs baseline: 1.3995x; 1.0269x over previous
import functools

import jax
import jax.numpy as jnp
from jax import lax
from jax.experimental import pallas as pl
from jax.experimental.pallas import tpu as pltpu
from jax.experimental.pallas import tpu_sc as plsc


def _sc_gather(t, emb):
    T = t.shape[0]
    V, D = emb.shape
    info = plsc.get_sparse_core_info()
    nc, ns = info.num_cores, info.num_subcores
    nw = nc * ns
    rows_per_w = T // nw

    mesh = plsc.VectorSubcoreMesh(core_axis_name="c", subcore_axis_name="s")

    @functools.partial(
        pl.kernel,
        mesh=mesh,
        out_type=jax.ShapeDtypeStruct((T, D), jnp.float32),
        scratch_types=[
            pltpu.VMEM((rows_per_w,), jnp.int32),
            pltpu.VMEM((rows_per_w, D), jnp.float32),
            pltpu.SemaphoreType.DMA,
        ],
    )
    def gather_kernel(t_hbm, emb_hbm, out_hbm, idx_v, rows_v, sem):
        wid = lax.axis_index("s") * nc + lax.axis_index("c")
        base = wid * rows_per_w
        pltpu.sync_copy(t_hbm.at[pl.ds(base, rows_per_w)], idx_v)
        pltpu.async_copy(emb_hbm.at[idx_v], rows_v, sem).wait()
        pltpu.sync_copy(rows_v, out_hbm.at[pl.ds(base, rows_per_w)])

    return gather_kernel(t, emb)


_TR = 512


def _tc_body(x_ref, e_ref, w_ref, b_ref, out_ref, te_ref, w16_ref):
    i = pl.program_id(0)
    j = pl.program_id(1)
    nt = pl.num_programs(1)

    def _mm(blk):
        rows = e_ref[pl.ds(blk * _TR, _TR), :].astype(jnp.bfloat16)
        return lax.dot_general(
            rows, w16_ref[...],
            dimension_numbers=(((1,), (1,)), ((), ())),
            preferred_element_type=jnp.float32,
        ) + b_ref[...]

    @pl.when((i == 0) & (j == 0))
    def _():
        w16_ref[...] = w_ref[...].astype(jnp.bfloat16)
        te_ref[pl.ds(0, _TR), :] = _mm(0)
        te_ref[pl.ds(_TR, _TR), :] = _mm(1)

    @pl.when((i == 0) & (j >= 1) & (j < nt - 1))
    def _():
        blk = j + 1
        te_ref[pl.ds(blk * _TR, _TR), :] = _mm(blk)

    out_ref[...] = x_ref[...] + te_ref[pl.ds(j * _TR, _TR), :][None]


def _proj_add(x3, e, W, b, interpret=False):
    BH, T, D = x3.shape
    nt = T // _TR
    b2 = b.reshape(1, D)
    return pl.pallas_call(
        _tc_body,
        grid=(BH, nt),
        in_specs=[
            pl.BlockSpec((1, _TR, D), lambda i, j: (i, j, 0)),
            pl.BlockSpec((T, D), lambda i, j: (0, 0)),
            pl.BlockSpec((D, D), lambda i, j: (0, 0)),
            pl.BlockSpec((1, D), lambda i, j: (0, 0)),
        ],
        out_specs=pl.BlockSpec((1, _TR, D), lambda i, j: (i, j, 0)),
        out_shape=jax.ShapeDtypeStruct((BH, T, D), jnp.float32),
        scratch_shapes=[
            pltpu.VMEM((T, D), jnp.float32),
            pltpu.VMEM((D, D), jnp.bfloat16),
        ],
        interpret=interpret,
    )(x3, e, W, b2)


def kernel(x, t, emb, W, b):
    B, H, T, D = x.shape
    e = _sc_gather(t, emb)
    x3 = x.reshape(B * H, T, D)
    out3 = _proj_add(x3, e, W, b)
    return out3.reshape(B, H, T, D)


# speedup vs baseline: 1.5145x; 1.0821x over previous
import functools

import jax
import jax.numpy as jnp
from jax import lax
from jax.experimental import pallas as pl
from jax.experimental.pallas import tpu as pltpu
from jax.experimental.pallas import tpu_sc as plsc


def _sc_gather(t, emb):
    T = t.shape[0]
    V, D = emb.shape
    info = plsc.get_sparse_core_info()
    nc, ns = info.num_cores, info.num_subcores
    nw = nc * ns
    rows_per_w = T // nw

    mesh = plsc.VectorSubcoreMesh(core_axis_name="c", subcore_axis_name="s")

    @functools.partial(
        pl.kernel,
        mesh=mesh,
        out_type=jax.ShapeDtypeStruct((T, D), jnp.float32),
        scratch_types=[
            pltpu.VMEM((rows_per_w,), jnp.int32),
            pltpu.VMEM((rows_per_w, D), jnp.float32),
            pltpu.SemaphoreType.DMA,
        ],
    )
    def gather_kernel(t_hbm, emb_hbm, out_hbm, idx_v, rows_v, sem):
        wid = lax.axis_index("s") * nc + lax.axis_index("c")
        base = wid * rows_per_w
        pltpu.sync_copy(t_hbm.at[pl.ds(base, rows_per_w)], idx_v)
        pltpu.async_copy(emb_hbm.at[idx_v], rows_v, sem).wait()
        pltpu.sync_copy(rows_v, out_hbm.at[pl.ds(base, rows_per_w)])

    return gather_kernel(t, emb)


_TR = 1024


def _tc_body(x_ref, e_ref, w_ref, b_ref, out_ref, te_ref, w16_ref):
    i = pl.program_id(0)
    j = pl.program_id(1)
    nt = pl.num_programs(1)

    def _mm(blk):
        rows = e_ref[pl.ds(blk * _TR, _TR), :].astype(jnp.bfloat16)
        return lax.dot_general(
            rows, w16_ref[...],
            dimension_numbers=(((1,), (1,)), ((), ())),
            preferred_element_type=jnp.float32,
        ) + b_ref[...]

    @pl.when((i == 0) & (j == 0))
    def _():
        w16_ref[...] = w_ref[...].astype(jnp.bfloat16)
        te_ref[pl.ds(0, _TR), :] = _mm(0)
        te_ref[pl.ds(_TR, _TR), :] = _mm(1)

    @pl.when((i == 0) & (j >= 1) & (j < nt - 1))
    def _():
        blk = j + 1
        te_ref[pl.ds(blk * _TR, _TR), :] = _mm(blk)

    out_ref[...] = x_ref[...] + te_ref[pl.ds(j * _TR, _TR), :][None]


def _proj_add(x3, e, W, b, interpret=False):
    BH, T, D = x3.shape
    nt = T // _TR
    b2 = b.reshape(1, D)
    return pl.pallas_call(
        _tc_body,
        grid=(BH, nt),
        in_specs=[
            pl.BlockSpec((1, _TR, D), lambda i, j: (i, j, 0)),
            pl.BlockSpec((T, D), lambda i, j: (0, 0)),
            pl.BlockSpec((D, D), lambda i, j: (0, 0)),
            pl.BlockSpec((1, D), lambda i, j: (0, 0)),
        ],
        out_specs=pl.BlockSpec((1, _TR, D), lambda i, j: (i, j, 0)),
        out_shape=jax.ShapeDtypeStruct((BH, T, D), jnp.float32),
        scratch_shapes=[
            pltpu.VMEM((T, D), jnp.float32),
            pltpu.VMEM((D, D), jnp.bfloat16),
        ],
        interpret=interpret,
    )(x3, e, W, b2)


def kernel(x, t, emb, W, b):
    B, H, T, D = x.shape
    e = _sc_gather(t, emb)
    x3 = x.reshape(B * H, T, D)
    out3 = _proj_add(x3, e, W, b)
    return out3.reshape(B, H, T, D)


# speedup vs baseline: 1.5445x; 1.0198x over previous
import functools

import jax
import jax.numpy as jnp
from jax import lax
from jax.experimental import pallas as pl
from jax.experimental.pallas import tpu as pltpu
from jax.experimental.pallas import tpu_sc as plsc


def _sc_gather(t, emb):
    T = t.shape[0]
    V, D = emb.shape
    info = plsc.get_sparse_core_info()
    nc, ns = info.num_cores, info.num_subcores
    nw = nc * ns
    rows_per_w = T // nw

    mesh = plsc.VectorSubcoreMesh(core_axis_name="c", subcore_axis_name="s")

    @functools.partial(
        pl.kernel,
        mesh=mesh,
        out_type=jax.ShapeDtypeStruct((T, D), jnp.float32),
        scratch_types=[
            pltpu.VMEM((rows_per_w,), jnp.int32),
            pltpu.VMEM((rows_per_w, D), jnp.float32),
            pltpu.SemaphoreType.DMA,
        ],
    )
    def gather_kernel(t_hbm, emb_hbm, out_hbm, idx_v, rows_v, sem):
        wid = lax.axis_index("s") * nc + lax.axis_index("c")
        base = wid * rows_per_w
        pltpu.sync_copy(t_hbm.at[pl.ds(base, rows_per_w)], idx_v)
        pltpu.async_copy(emb_hbm.at[idx_v], rows_v, sem).wait()
        pltpu.sync_copy(rows_v, out_hbm.at[pl.ds(base, rows_per_w)])

    return gather_kernel(t, emb)


_TR = 512
_BG = 4


def _tc_body(x_ref, e_ref, w16_ref, b_ref, out_ref, te_ref):
    i = pl.program_id(0)
    j = pl.program_id(1)
    nt = pl.num_programs(1)

    def _mm(blk):
        rows = e_ref[pl.ds(blk * _TR, _TR), :].astype(jnp.bfloat16)
        return lax.dot_general(
            rows, w16_ref[...],
            dimension_numbers=(((1,), (1,)), ((), ())),
            preferred_element_type=jnp.float32,
        ) + b_ref[...]

    @pl.when((i == 0) & (j == 0))
    def _():
        te_ref[pl.ds(0, _TR), :] = _mm(0)
        te_ref[pl.ds(_TR, _TR), :] = _mm(1)

    @pl.when((i == 0) & (j >= 1) & (j < nt - 1))
    def _():
        blk = j + 1
        te_ref[pl.ds(blk * _TR, _TR), :] = _mm(blk)

    out_ref[...] = x_ref[...] + te_ref[pl.ds(j * _TR, _TR), :][None]


def _proj_add(x3, e, W16, b, interpret=False):
    BH, T, D = x3.shape
    nt = T // _TR
    b2 = b.reshape(1, D)
    return pl.pallas_call(
        _tc_body,
        grid=(BH // _BG, nt),
        in_specs=[
            pl.BlockSpec((_BG, _TR, D), lambda i, j: (i, j, 0)),
            pl.BlockSpec((T, D), lambda i, j: (0, 0)),
            pl.BlockSpec((D, D), lambda i, j: (0, 0)),
            pl.BlockSpec((1, D), lambda i, j: (0, 0)),
        ],
        out_specs=pl.BlockSpec((_BG, _TR, D), lambda i, j: (i, j, 0)),
        out_shape=jax.ShapeDtypeStruct((BH, T, D), jnp.float32),
        scratch_shapes=[
            pltpu.VMEM((T, D), jnp.float32),
        ],
        interpret=interpret,
    )(x3, e, W16, b2)


def kernel(x, t, emb, W, b):
    B, H, T, D = x.shape
    e = _sc_gather(t, emb)
    x3 = x.reshape(B * H, T, D)
    out3 = _proj_add(x3, e, W.astype(jnp.bfloat16), b)
    return out3.reshape(B, H, T, D)


# speedup vs baseline: 1.5526x; 1.0052x over previous
import functools

import jax
import jax.numpy as jnp
from jax import lax
from jax.experimental import pallas as pl
from jax.experimental.pallas import tpu as pltpu
from jax.experimental.pallas import tpu_sc as plsc


def _sc_gather(t, emb):
    T = t.shape[0]
    V, D = emb.shape
    info = plsc.get_sparse_core_info()
    nc, ns = info.num_cores, info.num_subcores
    nw = nc * ns
    rows_per_w = T // nw

    mesh = plsc.VectorSubcoreMesh(core_axis_name="c", subcore_axis_name="s")

    @functools.partial(
        pl.kernel,
        mesh=mesh,
        out_type=jax.ShapeDtypeStruct((T, D), jnp.float32),
        scratch_types=[
            pltpu.VMEM((rows_per_w,), jnp.int32),
            pltpu.VMEM((rows_per_w, D), jnp.float32),
            pltpu.SemaphoreType.DMA,
        ],
    )
    def gather_kernel(t_hbm, emb_hbm, out_hbm, idx_v, rows_v, sem):
        wid = lax.axis_index("s") * nc + lax.axis_index("c")
        base = wid * rows_per_w
        pltpu.sync_copy(t_hbm.at[pl.ds(base, rows_per_w)], idx_v)
        pltpu.async_copy(emb_hbm.at[idx_v], rows_v, sem).wait()
        pltpu.sync_copy(rows_v, out_hbm.at[pl.ds(base, rows_per_w)])

    return gather_kernel(t, emb)


_TR = 256
_BG = 8


def _tc_body(x_ref, e_ref, w16_ref, b_ref, out_ref, te_ref):
    i = pl.program_id(0)
    j = pl.program_id(1)
    nt = pl.num_programs(1)

    def _mm(blk):
        rows = e_ref[pl.ds(blk * _TR, _TR), :].astype(jnp.bfloat16)
        return lax.dot_general(
            rows, w16_ref[...],
            dimension_numbers=(((1,), (1,)), ((), ())),
            preferred_element_type=jnp.float32,
        ) + b_ref[...]

    @pl.when((i == 0) & (j == 0))
    def _():
        te_ref[pl.ds(0, _TR), :] = _mm(0)
        te_ref[pl.ds(_TR, _TR), :] = _mm(1)

    @pl.when((i == 0) & (j >= 1) & (j < nt - 1))
    def _():
        blk = j + 1
        te_ref[pl.ds(blk * _TR, _TR), :] = _mm(blk)

    out_ref[...] = x_ref[...] + te_ref[pl.ds(j * _TR, _TR), :][None]


def _proj_add(x3, e, W16, b, interpret=False):
    BH, T, D = x3.shape
    nt = T // _TR
    b2 = b.reshape(1, D)
    return pl.pallas_call(
        _tc_body,
        grid=(BH // _BG, nt),
        in_specs=[
            pl.BlockSpec((_BG, _TR, D), lambda i, j: (i, j, 0)),
            pl.BlockSpec((T, D), lambda i, j: (0, 0)),
            pl.BlockSpec((D, D), lambda i, j: (0, 0)),
            pl.BlockSpec((1, D), lambda i, j: (0, 0)),
        ],
        out_specs=pl.BlockSpec((_BG, _TR, D), lambda i, j: (i, j, 0)),
        out_shape=jax.ShapeDtypeStruct((BH, T, D), jnp.float32),
        scratch_shapes=[
            pltpu.VMEM((T, D), jnp.float32),
        ],
        interpret=interpret,
    )(x3, e, W16, b2)


def kernel(x, t, emb, W, b):
    B, H, T, D = x.shape
    e = _sc_gather(t, emb)
    x3 = x.reshape(B * H, T, D)
    out3 = _proj_add(x3, e, W.astype(jnp.bfloat16), b)
    return out3.reshape(B, H, T, D)


# speedup vs baseline: 1.5612x; 1.0055x over previous
import functools

import jax
import jax.numpy as jnp
from jax import lax
from jax.experimental import pallas as pl
from jax.experimental.pallas import tpu as pltpu
from jax.experimental.pallas import tpu_sc as plsc


def _sc_gather(t, emb):
    T = t.shape[0]
    V, D = emb.shape
    info = plsc.get_sparse_core_info()
    nc, ns = info.num_cores, info.num_subcores
    nw = nc * ns
    rows_per_w = T // nw

    mesh = plsc.VectorSubcoreMesh(core_axis_name="c", subcore_axis_name="s")

    @functools.partial(
        pl.kernel,
        mesh=mesh,
        out_type=jax.ShapeDtypeStruct((T, D), jnp.float32),
        scratch_types=[
            pltpu.VMEM((rows_per_w,), jnp.int32),
            pltpu.VMEM((rows_per_w, D), jnp.float32),
            pltpu.SemaphoreType.DMA,
        ],
    )
    def gather_kernel(t_hbm, emb_hbm, out_hbm, idx_v, rows_v, sem):
        wid = lax.axis_index("s") * nc + lax.axis_index("c")
        base = wid * rows_per_w
        pltpu.sync_copy(t_hbm.at[pl.ds(base, rows_per_w)], idx_v)
        pltpu.async_copy(emb_hbm.at[idx_v], rows_v, sem).wait()
        pltpu.sync_copy(rows_v, out_hbm.at[pl.ds(base, rows_per_w)])

    return gather_kernel(t, emb)


_TR = 256
_BG = 8


def _tc_body(x_ref, e_ref, w16_ref, b_ref, out_ref, te_ref):
    i = pl.program_id(0)
    j = pl.program_id(1)

    @pl.when(i == 0)
    def _():
        rows = e_ref[...].astype(jnp.bfloat16)
        te_ref[pl.ds(j * _TR, _TR), :] = lax.dot_general(
            rows, w16_ref[...],
            dimension_numbers=(((1,), (1,)), ((), ())),
            preferred_element_type=jnp.float32,
        ) + b_ref[...]

    out_ref[...] = x_ref[...] + te_ref[pl.ds(j * _TR, _TR), :][None]


def _proj_add(x3, e, W16, b, interpret=False):
    BH, T, D = x3.shape
    nt = T // _TR
    b2 = b.reshape(1, D)
    return pl.pallas_call(
        _tc_body,
        grid=(BH // _BG, nt),
        in_specs=[
            pl.BlockSpec((_BG, _TR, D), lambda i, j: (i, j, 0)),
            pl.BlockSpec((_TR, D),
                         lambda i, j: (jnp.where(i == 0, j, nt - 1), 0)),
            pl.BlockSpec((D, D), lambda i, j: (0, 0)),
            pl.BlockSpec((1, D), lambda i, j: (0, 0)),
        ],
        out_specs=pl.BlockSpec((_BG, _TR, D), lambda i, j: (i, j, 0)),
        out_shape=jax.ShapeDtypeStruct((BH, T, D), jnp.float32),
        scratch_shapes=[
            pltpu.VMEM((T, D), jnp.float32),
        ],
        interpret=interpret,
    )(x3, e, W16, b2)


def kernel(x, t, emb, W, b):
    B, H, T, D = x.shape
    e = _sc_gather(t, emb)
    x3 = x.reshape(B * H, T, D)
    out3 = _proj_add(x3, e, W.astype(jnp.bfloat16), b)
    return out3.reshape(B, H, T, D)
